# Initial kernel scaffold; baseline (speedup 1.0000x reference)
import jax
import jax.numpy as jnp
from jax.experimental import pallas as pl


def kernel(log_weight_K, embeddings_KxtxD, log_felsensteins_KxtxSxA, leaf_counts_Kxt, W_pair, W_branch, W_merge, W_q, W_stat):
    raise NotImplementedError("write your pallas kernel here")


# speedup vs baseline: 1.1418x; 1.1418x over previous
import jax
import jax.numpy as jnp
from jax.experimental import pallas as pl
from jax.experimental.pallas import tpu as pltpu
from jax.scipy.linalg import expm

_K, _T, _S, _A, _D = 128, 32, 512, 4, 64


def _lik_kernel(s_ref, logf_ref, lp1_ref, lp2_ref, ls_ref, mls_ref, out_ref):
    k = pl.program_id(0)
    i1 = s_ref[1, k]
    i2 = s_ref[2, k]
    f = logf_ref[0]
    ls = ls_ref[0]
    x = f + ls[:, :, None]
    m = jnp.max(x, axis=0)
    lse = m + jnp.log(jnp.sum(jnp.exp(x - m[None, :, :]), axis=0))
    tidx = jax.lax.broadcasted_iota(jnp.int32, (_T, _S), 0)
    keep = (tidx != i1) & (tidx != i2)
    total = jnp.sum(jnp.where(keep, lse, 0.0))
    f1 = logf_ref[0, :, pl.ds(i1, 1), :][:, 0, :]
    f2 = logf_ref[0, :, pl.ds(i2, 1), :][:, 0, :]
    lp1 = lp1_ref[0]
    lp2 = lp2_ref[0]
    z1 = lp1[:, :, None] + f1[None, :, :]
    m1 = jnp.max(z1, axis=1)
    l1 = m1 + jnp.log(jnp.sum(jnp.exp(z1 - m1[:, None, :]), axis=1))
    z2 = lp2[:, :, None] + f2[None, :, :]
    m2 = jnp.max(z2, axis=1)
    l2 = m2 + jnp.log(jnp.sum(jnp.exp(z2 - m2[:, None, :]), axis=1))
    nf = l1 + l2
    mls = mls_ref[0, 0]
    w = nf + mls[:, None]
    mw = jnp.max(w, axis=0)
    mlik = jnp.sum(mw + jnp.log(jnp.sum(jnp.exp(w - mw[None, :]), axis=0)))
    out_ref[pl.ds(k, 1), :] = jnp.reshape(total + mlik, (1, 1))


def kernel(log_weight_K, embeddings_KxtxD, log_felsensteins_KxtxSxA,
           leaf_counts_Kxt, W_pair, W_branch, W_merge, W_q, W_stat):
    key = jax.random.key(42)
    indexes_K = jax.random.categorical(
        jax.random.fold_in(key, 0), log_weight_K, shape=(_K,))
    emb_KxtxD = jnp.take(embeddings_KxtxD, indexes_K, axis=0)
    scores_Kxt = emb_KxtxD @ W_pair
    u = jax.random.uniform(jax.random.fold_in(key, 1), scores_Kxt.shape,
                           minval=1e-6, maxval=1.0 - 1e-6)
    g = -jnp.log(-jnp.log(u))
    _, top_idx = jax.lax.top_k(scores_Kxt + g, 2)
    idx1_K = top_idx[:, 0]
    idx2_K = top_idx[:, 1]
    rows = jnp.arange(_K)
    emb1_KxD = emb_KxtxD[rows, idx1_K]
    emb2_KxD = emb_KxtxD[rows, idx2_K]
    pair_Kx2D = jnp.concatenate([emb1_KxD, emb2_KxD], axis=-1)
    b_Kx2 = jax.nn.softplus((emb1_KxD + emb2_KxD) @ W_branch) + 1e-4
    branch1_K = b_Kx2[:, 0]
    branch2_K = b_Kx2[:, 1]
    merged_emb_KxD = jnp.tanh(pair_Kx2D @ W_merge)
    raw_KxAxA = (merged_emb_KxD @ W_q).reshape(_K, _A, _A)
    eye_AxA = jnp.eye(_A, dtype=raw_KxAxA.dtype)
    off_KxAxA = jax.nn.softplus(raw_KxAxA) * (1.0 - eye_AxA)
    Q_KxAxA = off_KxAxA - eye_AxA * off_KxAxA.sum(-1, keepdims=True)
    P1_KxAxA = jax.vmap(expm)(Q_KxAxA * branch1_K[:, None, None])
    P2_KxAxA = jax.vmap(expm)(Q_KxAxA * branch2_K[:, None, None])
    logP1 = jnp.log(jnp.clip(P1_KxAxA, 1e-30))
    logP2 = jnp.log(jnp.clip(P2_KxAxA, 1e-30))
    ls_KxtxA = jnp.log(jnp.clip(jax.nn.softmax(emb_KxtxD @ W_stat, -1), 1e-30))
    lsT_KxAxt = jnp.transpose(ls_KxtxA, (0, 2, 1))
    mls_Kx1xA = jnp.log(jnp.clip(
        jax.nn.softmax(merged_emb_KxD @ W_stat, -1), 1e-30))[:, None, :]
    logf_KxAxtxS = jnp.transpose(log_felsensteins_KxtxSxA, (0, 3, 1, 2))
    sidx_3xK = jnp.stack([indexes_K, idx1_K, idx2_K]).astype(jnp.int32)

    grid_spec = pltpu.PrefetchScalarGridSpec(
        num_scalar_prefetch=1,
        grid=(_K,),
        in_specs=[
            pl.BlockSpec((1, _A, _T, _S), lambda k, s: (s[0, k], 0, 0, 0)),
            pl.BlockSpec((1, _A, _A), lambda k, s: (k, 0, 0)),
            pl.BlockSpec((1, _A, _A), lambda k, s: (k, 0, 0)),
            pl.BlockSpec((1, _A, _T), lambda k, s: (k, 0, 0)),
            pl.BlockSpec((1, 1, _A), lambda k, s: (k, 0, 0)),
        ],
        out_specs=pl.BlockSpec((_K, 1), lambda k, s: (0, 0)),
    )
    log_likelihood_K = pl.pallas_call(
        _lik_kernel,
        grid_spec=grid_spec,
        out_shape=jax.ShapeDtypeStruct((_K, 1), jnp.float32),
    )(sidx_3xK, logf_KxAxtxS, logP1, logP2, lsT_KxAxt, mls_Kx1xA)[:, 0]

    log_prior_K = 2.0 * jnp.log(10.0) - 10.0 * (branch1_K + branch2_K)
    log_v_plus = jnp.log(_T * (_T - 1) / 2.0)
    return log_likelihood_K + log_prior_K + log_v_plus
